# Initial kernel scaffold; baseline (speedup 1.0000x reference)
import functools

import jax
import jax.numpy as jnp
from jax import lax
from jax.experimental import pallas as pl
from jax.experimental.pallas import tpu as pltpu
from jax.experimental.pallas import tpu_sc as plsc

_N = 10000
_E = 320000
_H = 128
_G = 128

_NC = 2
_NS = 16
_NW = _NC * _NS
_K = 80
_EW = _E // _NW
_NCHUNK = _EW // _K
_ZROWS = 125
_RPS = _N // _NS


def _sc_agg_body(h_hbm, ea_hbm, src_hbm, dst_hbm, out_hbm,
                 sidx, didx, hbuf, ebuf, zbuf, agg, sem):
    c = lax.axis_index("c")
    s = lax.axis_index("s")
    w = c * _NS + s

    zero = jnp.zeros((16,), jnp.float32)

    def zrow(i, carry):
        for j in range(_H // 16):
            zbuf[i, pl.ds(j * 16, 16)] = zero
        return carry

    lax.fori_loop(0, _ZROWS, zrow, 0)
    for j in range(_RPS // _ZROWS):
        pltpu.sync_copy(zbuf, agg.at[pl.ds(s * _RPS + j * _ZROWS, _ZROWS)])
    plsc.subcore_barrier()

    def step(i, carry):
        base = w * _EW + i * _K
        pltpu.sync_copy(src_hbm.at[pl.ds(base, _K)], sidx)
        pltpu.sync_copy(dst_hbm.at[pl.ds(base, _K)], didx)
        cp = pltpu.async_copy(h_hbm.at[sidx], hbuf, sem)
        pltpu.sync_copy(ea_hbm.at[pl.ds(base, _K)], ebuf)
        cp.wait()

        def row(k, rc):
            for j in range(_H // 16):
                sl = pl.ds(j * 16, 16)
                hbuf[k, sl] = jnp.maximum(hbuf[k, sl] + ebuf[k, sl], 0.0)
            return rc

        lax.fori_loop(0, _K, row, 0)
        pltpu.sync_copy(hbuf, agg.at[didx], add=True)
        return carry

    lax.fori_loop(0, _NCHUNK, step, 0)
    plsc.subcore_barrier()
    pltpu.sync_copy(agg.at[pl.ds(s * _RPS, _RPS)],
                    out_hbm.at[c, pl.ds(s * _RPS, _RPS)])


_sc_agg = pl.kernel(
    _sc_agg_body,
    out_type=jax.ShapeDtypeStruct((_NC, _N, _H), jnp.float32),
    mesh=plsc.VectorSubcoreMesh(core_axis_name="c", subcore_axis_name="s",
                                num_cores=_NC, num_subcores=_NS),
    scratch_types=[
        pltpu.VMEM((_K,), jnp.int32),
        pltpu.VMEM((_K,), jnp.int32),
        pltpu.VMEM((_K, _H), jnp.float32),
        pltpu.VMEM((_K, _H), jnp.float32),
        pltpu.VMEM((_ZROWS, _H), jnp.float32),
        pltpu.VMEM_SHARED((_N, _H), jnp.float32),
        pltpu.SemaphoreType.DMA,
    ],
)


def _edge_proj_body(attr_ref, we_ref, be_ref, out_ref):
    out_ref[...] = (
        jnp.dot(attr_ref[...], we_ref[...], preferred_element_type=jnp.float32)
        + be_ref[...]
    )


def _edge_proj(edge_attr, We, be):
    eb = 4000
    grid = _E // eb
    return pl.pallas_call(
        _edge_proj_body,
        grid=(grid,),
        in_specs=[
            pl.BlockSpec((eb, 16), lambda i: (i, 0)),
            pl.BlockSpec((16, _H), lambda i: (0, 0)),
            pl.BlockSpec((1, _H), lambda i: (0, 0)),
        ],
        out_specs=pl.BlockSpec((eb, _H), lambda i: (i, 0)),
        out_shape=jax.ShapeDtypeStruct((_E, _H), jnp.float32),
    )(edge_attr, We, be.reshape(1, _H))


def _node_proj_body(x_ref, wn_ref, bn_ref, out_ref):
    out_ref[...] = (
        jnp.dot(x_ref[...], wn_ref[...], preferred_element_type=jnp.float32)
        + bn_ref[...]
    )


def _node_proj(x, Wn, bn_):
    return pl.pallas_call(
        _node_proj_body,
        out_shape=jax.ShapeDtypeStruct((_N, _H), jnp.float32),
    )(x, Wn, bn_.reshape(1, _H))


def _layer_body(h_ref, parts_ref, w1_ref, b1_ref, w2_ref, b2_ref,
                gm_ref, bt_ref, eps_ref, out_ref):
    h = h_ref[...]
    h2 = (1.0 + eps_ref[0]) * h + parts_ref[0] + parts_ref[1]
    a = jnp.maximum(
        jnp.dot(h2, w1_ref[...], preferred_element_type=jnp.float32)
        + b1_ref[...], 0.0)
    z = (jnp.dot(a, w2_ref[...], preferred_element_type=jnp.float32)
         + b2_ref[...])
    mu = jnp.mean(z, axis=0, keepdims=True)
    zc = z - mu
    var = jnp.mean(zc * zc, axis=0, keepdims=True)
    zn = zc * lax.rsqrt(var + 1e-5) * gm_ref[...] + bt_ref[...]
    out_ref[...] = jnp.maximum(zn, 0.0) + h


def _layer(h, parts, W1l, b1l, W2l, b2l, gml, btl, epsl):
    return pl.pallas_call(
        _layer_body,
        out_shape=jax.ShapeDtypeStruct((_N, _H), jnp.float32),
    )(h, parts, W1l, b1l.reshape(1, _H), W2l, b2l.reshape(1, _H),
      gml.reshape(1, _H), btl.reshape(1, _H), epsl.reshape(1))


def _head_body(h_ref, batch_ref, wc1_ref, bc1_ref, wc2_ref, bc2_ref,
               logits_ref, probs_ref, preds_ref):
    h = h_ref[...]
    b = batch_ref[...]
    onehot = (b == lax.broadcasted_iota(jnp.int32, (1, _G), 1)).astype(
        jnp.float32)
    sums = lax.dot_general(onehot, h, (((0,), (0,)), ((), ())),
                           preferred_element_type=jnp.float32)
    cnts = jnp.sum(onehot, axis=0, keepdims=True)
    g = sums / jnp.maximum(cnts, 1.0).reshape(_G, 1)
    gh = jnp.maximum(
        jnp.dot(g, wc1_ref[...], preferred_element_type=jnp.float32)
        + bc1_ref[...], 0.0)
    logits = (jnp.dot(gh, wc2_ref[...], preferred_element_type=jnp.float32)
              + bc2_ref[...])
    probs = 1.0 / (1.0 + jnp.exp(-logits))
    logits_ref[...] = logits
    probs_ref[...] = probs
    preds_ref[...] = (probs > 0.5).astype(jnp.float32)


def _head(h, batch, Wc1, bc1, Wc2, bc2):
    c = Wc2.shape[1]
    return pl.pallas_call(
        _head_body,
        out_shape=(
            jax.ShapeDtypeStruct((_G, c), jnp.float32),
            jax.ShapeDtypeStruct((_G, c), jnp.float32),
            jax.ShapeDtypeStruct((_G, c), jnp.float32),
        ),
    )(h, batch.reshape(_N, 1), Wc1, bc1.reshape(1, _H), Wc2,
      bc2.reshape(1, c))


def kernel(x, edge_index, batch, edge_attr, Wn, bn_, We, be, eps,
           W1, b1, W2, b2, gamma, beta, Wc1, bc1, Wc2, bc2):
    src = edge_index[0]
    dst = edge_index[1]
    h = _node_proj(x, Wn, bn_)
    ea = _edge_proj(edge_attr, We, be)
    num_layers = W1.shape[0]
    for l in range(num_layers):
        parts = _sc_agg(h, ea, src, dst)
        h = _layer(h, parts, W1[l], b1[l], W2[l], b2[l],
                   gamma[l], beta[l], eps[l])
    logits, probs, preds = _head(h, batch, Wc1, bc1, Wc2, bc2)
    return (logits, probs, preds, preds)


# speedup vs baseline: 3.3071x; 3.3071x over previous
import functools

import jax
import jax.numpy as jnp
from jax import lax
from jax.experimental import pallas as pl
from jax.experimental.pallas import tpu as pltpu
from jax.experimental.pallas import tpu_sc as plsc

_N = 10000
_E = 320000
_H = 128
_G = 128

_NC = 2
_NS = 16
_NW = _NC * _NS
_K = 80
_EW = _E // _NW
_NCHUNK = _EW // _K
_ZROWS = 80
_NZCH = _N // _ZROWS


def _sc_agg_body(h_hbm, ea_hbm, src_hbm, dst_hbm, out_hbm,
                 sidx, didx, hbuf, ebuf, zbuf, agg, sem):
    c = lax.axis_index("c")
    s = lax.axis_index("s")
    w = c * _NS + s

    zero = jnp.zeros((16,), jnp.float32)

    def zrow(i, carry):
        for j in range(_H // 16):
            zbuf[i, pl.ds(j * 16, 16)] = zero
        return carry

    lax.fori_loop(0, _ZROWS, zrow, 0)
    nz = jnp.where(s <= (_NZCH % _NS) - 1, _NZCH // _NS + 1, _NZCH // _NS)

    def zcopy(i, carry):
        r = (s + _NS * i) * _ZROWS
        pltpu.sync_copy(zbuf, agg.at[pl.ds(r, _ZROWS)])
        return carry

    lax.fori_loop(0, nz, zcopy, 0)
    plsc.subcore_barrier()

    def step(i, carry):
        base = w * _EW + i * _K
        pltpu.sync_copy(src_hbm.at[pl.ds(base, _K)], sidx)
        pltpu.sync_copy(dst_hbm.at[pl.ds(base, _K)], didx)
        cp = pltpu.async_copy(h_hbm.at[sidx], hbuf, sem)
        pltpu.sync_copy(ea_hbm.at[pl.ds(base, _K)], ebuf)
        cp.wait()

        def row(k, rc):
            for j in range(_H // 16):
                sl = pl.ds(j * 16, 16)
                hbuf[k, sl] = jnp.maximum(hbuf[k, sl] + ebuf[k, sl], 0.0)
            return rc

        lax.fori_loop(0, _K, row, 0)
        pltpu.sync_copy(hbuf, agg.at[didx], add=True)
        return carry

    lax.fori_loop(0, _NCHUNK, step, 0)
    plsc.subcore_barrier()

    def wcopy(i, carry):
        r = (s + _NS * i) * _ZROWS
        pltpu.sync_copy(agg.at[pl.ds(r, _ZROWS)],
                        out_hbm.at[c, pl.ds(r, _ZROWS)])
        return carry

    lax.fori_loop(0, nz, wcopy, 0)


_sc_agg = pl.kernel(
    _sc_agg_body,
    out_type=jax.ShapeDtypeStruct((_NC, _N, _H), jnp.float32),
    mesh=plsc.VectorSubcoreMesh(core_axis_name="c", subcore_axis_name="s",
                                num_cores=_NC, num_subcores=_NS),
    scratch_types=[
        pltpu.VMEM((_K,), jnp.int32),
        pltpu.VMEM((_K,), jnp.int32),
        pltpu.VMEM((_K, _H), jnp.float32),
        pltpu.VMEM((_K, _H), jnp.float32),
        pltpu.VMEM((_ZROWS, _H), jnp.float32),
        pltpu.VMEM_SHARED((_N, _H), jnp.float32),
        pltpu.SemaphoreType.DMA,
    ],
)


def _edge_proj_body(attr_ref, we_ref, be_ref, out_ref):
    out_ref[...] = (
        jnp.dot(attr_ref[...], we_ref[...], preferred_element_type=jnp.float32)
        + be_ref[...]
    )


def _edge_proj(edge_attr, We, be):
    eb = 4000
    grid = _E // eb
    return pl.pallas_call(
        _edge_proj_body,
        grid=(grid,),
        in_specs=[
            pl.BlockSpec((eb, 16), lambda i: (i, 0)),
            pl.BlockSpec((16, _H), lambda i: (0, 0)),
            pl.BlockSpec((1, _H), lambda i: (0, 0)),
        ],
        out_specs=pl.BlockSpec((eb, _H), lambda i: (i, 0)),
        out_shape=jax.ShapeDtypeStruct((_E, _H), jnp.float32),
    )(edge_attr, We, be.reshape(1, _H))


def _node_proj_body(x_ref, wn_ref, bn_ref, out_ref):
    out_ref[...] = (
        jnp.dot(x_ref[...], wn_ref[...], preferred_element_type=jnp.float32)
        + bn_ref[...]
    )


def _node_proj(x, Wn, bn_):
    return pl.pallas_call(
        _node_proj_body,
        out_shape=jax.ShapeDtypeStruct((_N, _H), jnp.float32),
    )(x, Wn, bn_.reshape(1, _H))


def _layer_body(h_ref, parts_ref, w1_ref, b1_ref, w2_ref, b2_ref,
                gm_ref, bt_ref, eps_ref, out_ref):
    h = h_ref[...]
    h2 = (1.0 + eps_ref[0]) * h + parts_ref[0] + parts_ref[1]
    a = jnp.maximum(
        jnp.dot(h2, w1_ref[...], preferred_element_type=jnp.float32)
        + b1_ref[...], 0.0)
    z = (jnp.dot(a, w2_ref[...], preferred_element_type=jnp.float32)
         + b2_ref[...])
    mu = jnp.mean(z, axis=0, keepdims=True)
    zc = z - mu
    var = jnp.mean(zc * zc, axis=0, keepdims=True)
    zn = zc * lax.rsqrt(var + 1e-5) * gm_ref[...] + bt_ref[...]
    out_ref[...] = jnp.maximum(zn, 0.0) + h


def _layer(h, parts, W1l, b1l, W2l, b2l, gml, btl, epsl):
    return pl.pallas_call(
        _layer_body,
        out_shape=jax.ShapeDtypeStruct((_N, _H), jnp.float32),
    )(h, parts, W1l, b1l.reshape(1, _H), W2l, b2l.reshape(1, _H),
      gml.reshape(1, _H), btl.reshape(1, _H), epsl.reshape(1))


def _head_body(h_ref, batch_ref, wc1_ref, bc1_ref, wc2_ref, bc2_ref,
               logits_ref, probs_ref, preds_ref):
    h = h_ref[...]
    b = batch_ref[...]
    onehot = (b == lax.broadcasted_iota(jnp.int32, (1, _G), 1)).astype(
        jnp.float32)
    sums = lax.dot_general(onehot, h, (((0,), (0,)), ((), ())),
                           preferred_element_type=jnp.float32)
    cnts = jnp.sum(onehot, axis=0, keepdims=True)
    g = sums / jnp.maximum(cnts, 1.0).reshape(_G, 1)
    gh = jnp.maximum(
        jnp.dot(g, wc1_ref[...], preferred_element_type=jnp.float32)
        + bc1_ref[...], 0.0)
    logits = (jnp.dot(gh, wc2_ref[...], preferred_element_type=jnp.float32)
              + bc2_ref[...])
    probs = 1.0 / (1.0 + jnp.exp(-logits))
    logits_ref[...] = logits
    probs_ref[...] = probs
    preds_ref[...] = (probs > 0.5).astype(jnp.float32)


def _head(h, batch, Wc1, bc1, Wc2, bc2):
    c = Wc2.shape[1]
    return pl.pallas_call(
        _head_body,
        out_shape=(
            jax.ShapeDtypeStruct((_G, c), jnp.float32),
            jax.ShapeDtypeStruct((_G, c), jnp.float32),
            jax.ShapeDtypeStruct((_G, c), jnp.float32),
        ),
    )(h, batch.reshape(_N, 1), Wc1, bc1.reshape(1, _H), Wc2,
      bc2.reshape(1, c))


def kernel(x, edge_index, batch, edge_attr, Wn, bn_, We, be, eps,
           W1, b1, W2, b2, gamma, beta, Wc1, bc1, Wc2, bc2):
    src = edge_index[0]
    dst = edge_index[1]
    h = _node_proj(x, Wn, bn_)
    ea = _edge_proj(edge_attr, We, be)
    num_layers = W1.shape[0]
    for l in range(num_layers):
        parts = _sc_agg(h, ea, src, dst)
        h = _layer(h, parts, W1[l], b1[l], W2[l], b2[l],
                   gamma[l], beta[l], eps[l])
    logits, probs, preds = _head(h, batch, Wc1, bc1, Wc2, bc2)
    return (logits, probs, preds, preds)
